# baseline (speedup 1.0000x reference)
import jax
import jax.numpy as jnp
from jax.experimental import pallas as pl
from jax.experimental.pallas import tpu as pltpu

B, K, N = 256, 8, 128
EPS = 1e-12


def _loss_kernel(nn_ref, x_ref, out_ref):
    b = pl.program_id(0)
    m = nn_ref[b]
    x = x_ref[0]

    i3 = jax.lax.broadcasted_iota(jnp.int32, (K, N, N), 1)
    j3 = jax.lax.broadcasted_iota(jnp.int32, (K, N, N), 2)
    mask3 = (i3 < m) & (j3 < m)

    xc = jnp.maximum(x, EPS)
    s_full = jnp.where(mask3, xc, x)
    a_m = jnp.where(mask3, xc, 0.0)
    l_m = jnp.where(mask3, xc * jnp.log(xc), 0.0)

    c_full = jnp.maximum(jnp.sum(s_full, axis=1), EPS)
    r_full = jnp.maximum(jnp.sum(s_full, axis=2), EPS)
    d_col = jnp.sum(a_m, axis=1)
    u_row = jnp.sum(a_m, axis=2)
    s_col = jnp.sum(l_m, axis=1)
    t_row = jnp.sum(l_m, axis=2)

    jv = jax.lax.broadcasted_iota(jnp.int32, (K, N), 1)
    vmask = jv < m
    e_col = jnp.where(vmask, (d_col * jnp.log(c_full) - s_col) / c_full, 0.0)
    e_row = jnp.where(vmask, (u_row * jnp.log(r_full) - t_row) / r_full, 0.0)

    loss_b = (jnp.sum(e_col) + jnp.sum(e_row)) / (K * m.astype(jnp.float32))

    @pl.when(b == 0)
    def _init():
        out_ref[0, 0] = 0.0

    out_ref[0, 0] += loss_b / B


def kernel(perm_soft, n_nodes):
    nn = n_nodes.astype(jnp.int32)
    out = pl.pallas_call(
        _loss_kernel,
        grid_spec=pltpu.PrefetchScalarGridSpec(
            num_scalar_prefetch=1,
            grid=(B,),
            in_specs=[
                pl.BlockSpec((1, K, N, N), lambda b, nn_ref: (b, 0, 0, 0)),
            ],
            out_specs=pl.BlockSpec(
                (1, 1), lambda b, nn_ref: (0, 0), memory_space=pltpu.SMEM
            ),
        ),
        out_shape=jax.ShapeDtypeStruct((1, 1), jnp.float32),
    )(nn, perm_soft)
    return out[0, 0]
